# baseline (speedup 1.0000x reference)
import jax
import jax.numpy as jnp
from jax import lax
from jax.experimental import pallas as pl
from jax.experimental.pallas import tpu as pltpu
from jax.experimental.pallas import tpu_sc as plsc

DIM = 4096
N_EXPERTS = 64
TOPK = 8
BLK = 1024
N_TOKENS = 16384
N_WORKERS = 32
PIPE_TOKENS = 4096
TPW = PIPE_TOKENS // N_WORKERS
CHUNK = 128
LANES = 16


def _routing_block(x_ref, w_ref, b_ref, or_ref, os_ref):
    x = x_ref[...]
    w = w_ref[...]
    logits = jax.lax.dot_general(
        w, x, (((1,), (1,)), ((), ())),
        preferred_element_type=jnp.float32,
        precision=jax.lax.Precision.DEFAULT,
    )
    s = jax.nn.sigmoid(logits)
    or_ref[...] = s + b_ref[...]
    os_ref[...] = s


def _tc_routing_t(x, weight, b_col):
    return pl.pallas_call(
        _routing_block,
        grid=(PIPE_TOKENS // BLK,),
        in_specs=[
            pl.BlockSpec((BLK, DIM), lambda i: (i, 0)),
            pl.BlockSpec((N_EXPERTS, DIM), lambda i: (0, 0)),
            pl.BlockSpec((N_EXPERTS, 1), lambda i: (0, 0)),
        ],
        out_specs=[pl.BlockSpec((N_EXPERTS, BLK), lambda i: (0, i)),
                   pl.BlockSpec((N_EXPERTS, BLK), lambda i: (0, i))],
        out_shape=[jax.ShapeDtypeStruct((N_EXPERTS, PIPE_TOKENS), jnp.float32),
                   jax.ShapeDtypeStruct((N_EXPERTS, PIPE_TOKENS), jnp.float32)],
    )(x, weight, b_col)


def _sc_topk_kernel(routing_hbm, scores_hbm, ow_hbm, oi_hbm,
                    blkr_v, blks_v, ow_v, oi_v):
    wid = lax.axis_index("s") * 2 + lax.axis_index("c")
    base = wid * TPW

    def chunk_body(c, _):
        tok0 = base + c * CHUNK
        pltpu.sync_copy(routing_hbm.at[:, pl.ds(tok0, CHUNK)], blkr_v)
        pltpu.sync_copy(scores_hbm.at[:, pl.ds(tok0, CHUNK)], blks_v)

        def group_body(g, _):
            t0 = g * LANES
            rs = [jnp.full((LANES,), -3.0, jnp.float32) for _ in range(TOPK)]
            ss = [jnp.full((LANES,), 0.0, jnp.float32) for _ in range(TOPK)]
            is_ = [jnp.full((LANES,), 0, jnp.int32) for _ in range(TOPK)]
            for e in range(N_EXPERTS):
                vr = blkr_v[e, pl.ds(t0, LANES)]
                vs = blks_v[e, pl.ds(t0, LANES)]
                vi = jnp.full((LANES,), e, jnp.int32)
                for j in range(TOPK):
                    m = vr > rs[j]
                    nr = jnp.where(m, vr, rs[j])
                    ns = jnp.where(m, vs, ss[j])
                    ni = jnp.where(m, vi, is_[j])
                    vr = jnp.where(m, rs[j], vr)
                    vs = jnp.where(m, ss[j], vs)
                    vi = jnp.where(m, is_[j], vi)
                    rs[j], ss[j], is_[j] = nr, ns, ni
            total = ss[0]
            for j in range(1, TOPK):
                total = total + ss[j]
            inv = 1.0 / total
            for j in range(TOPK):
                ow_v[j, pl.ds(t0, LANES)] = ss[j] * inv
                oi_v[j, pl.ds(t0, LANES)] = is_[j]
            return 0

        lax.fori_loop(0, CHUNK // LANES, group_body, 0)
        pltpu.sync_copy(ow_v, ow_hbm.at[:, pl.ds(tok0, CHUNK)])
        pltpu.sync_copy(oi_v, oi_hbm.at[:, pl.ds(tok0, CHUNK)])
        return 0

    lax.fori_loop(0, TPW // CHUNK, chunk_body, 0)


def kernel(x, weight, expert_biases):
    b_col = expert_biases.reshape(N_EXPERTS, 1)
    sc = pl.kernel(
        _sc_topk_kernel,
        mesh=plsc.VectorSubcoreMesh(core_axis_name="c", subcore_axis_name="s"),
        out_type=[
            jax.ShapeDtypeStruct((TOPK, PIPE_TOKENS), jnp.float32),
            jax.ShapeDtypeStruct((TOPK, PIPE_TOKENS), jnp.int32),
        ],
        scratch_types=[
            pltpu.VMEM((N_EXPERTS, CHUNK), jnp.float32),
            pltpu.VMEM((N_EXPERTS, CHUNK), jnp.float32),
            pltpu.VMEM((TOPK, CHUNK), jnp.float32),
            pltpu.VMEM((TOPK, CHUNK), jnp.int32),
        ],
    )
    ws, iws = [], []
    for c in range(N_TOKENS // PIPE_TOKENS):
        xc = jax.lax.slice_in_dim(x, c * PIPE_TOKENS, (c + 1) * PIPE_TOKENS)
        routing_t, scores_t = _tc_routing_t(xc, weight, b_col)
        ow_t, oi_t = sc(routing_t, scores_t)
        ws.append(ow_t)
        iws.append(oi_t)
    ow = jnp.concatenate(ws, axis=1)
    oi = jnp.concatenate(iws, axis=1)
    return ow.T, oi.T


# speedup vs baseline: 2.2717x; 2.2717x over previous
import jax
import jax.numpy as jnp
from jax import lax
from jax.experimental import pallas as pl
from jax.experimental.pallas import tpu as pltpu
from jax.experimental.pallas import tpu_sc as plsc

DIM = 4096
N_EXPERTS = 64
TOPK = 8
BLK = 1024
N_TOKENS = 16384
N_WORKERS = 32
PIPE_TOKENS = 16384
TPW = PIPE_TOKENS // N_WORKERS
CHUNK = 128
LANES = 16


def _routing_block(x_ref, w_ref, b_ref, or_ref):
    x = x_ref[...]
    w = w_ref[...]
    logits = jax.lax.dot_general(
        w, x, (((1,), (1,)), ((), ())),
        preferred_element_type=jnp.float32,
        precision=jax.lax.Precision.DEFAULT,
    )
    or_ref[...] = jax.nn.sigmoid(logits) + b_ref[...]


def _tc_routing_t(x, weight, b_col):
    return pl.pallas_call(
        _routing_block,
        grid=(PIPE_TOKENS // BLK,),
        in_specs=[
            pl.BlockSpec((BLK, DIM), lambda i: (i, 0)),
            pl.BlockSpec((N_EXPERTS, DIM), lambda i: (0, 0)),
            pl.BlockSpec((N_EXPERTS, 1), lambda i: (0, 0)),
        ],
        out_specs=pl.BlockSpec((N_EXPERTS, BLK), lambda i: (0, i)),
        out_shape=jax.ShapeDtypeStruct((N_EXPERTS, PIPE_TOKENS), jnp.float32),
    )(x, weight, b_col)


def _sc_topk_kernel(routing_hbm, biasbc_hbm, ow_hbm, oi_hbm,
                    blkr_v, biasbc_v, ow_v, oi_v):
    wid = lax.axis_index("s") * 2 + lax.axis_index("c")
    base = wid * TPW
    pltpu.sync_copy(biasbc_hbm, biasbc_v)

    def chunk_body(c, _):
        tok0 = base + c * CHUNK
        pltpu.sync_copy(routing_hbm.at[:, pl.ds(tok0, CHUNK)], blkr_v)

        def group_body(g, _):
            t0 = g * LANES
            rs = [jnp.full((LANES,), -3.0, jnp.float32) for _ in range(TOPK)]
            ss = [jnp.full((LANES,), 0.0, jnp.float32) for _ in range(TOPK)]
            is_ = [jnp.full((LANES,), 0, jnp.int32) for _ in range(TOPK)]
            for e in range(N_EXPERTS):
                vr = blkr_v[e, pl.ds(t0, LANES)]
                vs = vr - biasbc_v[e, pl.ds(0, LANES)]
                vi = jnp.full((LANES,), e, jnp.int32)
                for j in range(TOPK):
                    m = vr > rs[j]
                    nr = jnp.where(m, vr, rs[j])
                    ns = jnp.where(m, vs, ss[j])
                    ni = jnp.where(m, vi, is_[j])
                    vr = jnp.where(m, rs[j], vr)
                    vs = jnp.where(m, ss[j], vs)
                    vi = jnp.where(m, is_[j], vi)
                    rs[j], ss[j], is_[j] = nr, ns, ni
            total = ss[0]
            for j in range(1, TOPK):
                total = total + ss[j]
            inv = 1.0 / total
            for j in range(TOPK):
                ow_v[j, pl.ds(t0, LANES)] = ss[j] * inv
                oi_v[j, pl.ds(t0, LANES)] = is_[j]
            return 0

        lax.fori_loop(0, CHUNK // LANES, group_body, 0)
        pltpu.sync_copy(ow_v, ow_hbm.at[:, pl.ds(tok0, CHUNK)])
        pltpu.sync_copy(oi_v, oi_hbm.at[:, pl.ds(tok0, CHUNK)])
        return 0

    lax.fori_loop(0, TPW // CHUNK, chunk_body, 0)


def kernel(x, weight, expert_biases):
    b_col = expert_biases.reshape(N_EXPERTS, 1)
    sc = pl.kernel(
        _sc_topk_kernel,
        mesh=plsc.VectorSubcoreMesh(core_axis_name="c", subcore_axis_name="s"),
        out_type=[
            jax.ShapeDtypeStruct((TOPK, PIPE_TOKENS), jnp.float32),
            jax.ShapeDtypeStruct((TOPK, PIPE_TOKENS), jnp.int32),
        ],
        scratch_types=[
            pltpu.VMEM((N_EXPERTS, CHUNK), jnp.float32),
            pltpu.VMEM((N_EXPERTS, LANES), jnp.float32),
            pltpu.VMEM((TOPK, CHUNK), jnp.float32),
            pltpu.VMEM((TOPK, CHUNK), jnp.int32),
        ],
    )
    routing_t = _tc_routing_t(x, weight, b_col)
    bias_bc = jnp.tile(expert_biases.reshape(N_EXPERTS, 1), (1, LANES))
    ow_t, oi_t = sc(routing_t, bias_bc)
    return ow_t.T, oi_t.T


# speedup vs baseline: 2.2977x; 1.0115x over previous
import jax
import jax.numpy as jnp
from jax import lax
from jax.experimental import pallas as pl
from jax.experimental.pallas import tpu as pltpu
from jax.experimental.pallas import tpu_sc as plsc

DIM = 4096
N_EXPERTS = 64
TOPK = 8
BLK = 1024
N_TOKENS = 16384
N_WORKERS = 32
PIPE_TOKENS = 16384
TPW = PIPE_TOKENS // N_WORKERS
CHUNK = 256
LANES = 16


def _routing_block(x_ref, w_ref, b_ref, or_ref):
    x = x_ref[...]
    w = w_ref[...]
    logits = jax.lax.dot_general(
        w, x, (((1,), (1,)), ((), ())),
        preferred_element_type=jnp.float32,
        precision=jax.lax.Precision.DEFAULT,
    )
    or_ref[...] = jax.nn.sigmoid(logits) + b_ref[...]


def _tc_routing_t(x, weight, b_col):
    return pl.pallas_call(
        _routing_block,
        grid=(PIPE_TOKENS // BLK,),
        in_specs=[
            pl.BlockSpec((BLK, DIM), lambda i: (i, 0)),
            pl.BlockSpec((N_EXPERTS, DIM), lambda i: (0, 0)),
            pl.BlockSpec((N_EXPERTS, 1), lambda i: (0, 0)),
        ],
        out_specs=pl.BlockSpec((N_EXPERTS, BLK), lambda i: (0, i)),
        out_shape=jax.ShapeDtypeStruct((N_EXPERTS, PIPE_TOKENS), jnp.float32),
    )(x, weight, b_col)


def _sc_topk_kernel(routing_hbm, biasbc_hbm, ow_hbm, oi_hbm,
                    blkr_v, biasbc_v, ow_v, oi_v):
    wid = lax.axis_index("s") * 2 + lax.axis_index("c")
    base = wid * TPW
    pltpu.sync_copy(biasbc_hbm, biasbc_v)

    def chunk_body(c, _):
        tok0 = base + c * CHUNK
        pltpu.sync_copy(routing_hbm.at[:, pl.ds(tok0, CHUNK)], blkr_v)

        def group_body(g, _):
            t0 = g * LANES
            rs = [jnp.full((LANES,), -3.0, jnp.float32) for _ in range(TOPK)]
            ss = [jnp.full((LANES,), 0.0, jnp.float32) for _ in range(TOPK)]
            is_ = [jnp.full((LANES,), 0, jnp.int32) for _ in range(TOPK)]
            for e in range(N_EXPERTS):
                vr = blkr_v[e, pl.ds(t0, LANES)]
                vs = vr - biasbc_v[e, pl.ds(0, LANES)]
                vi = jnp.full((LANES,), e, jnp.int32)
                for j in range(TOPK):
                    m = vr > rs[j]
                    nr = jnp.where(m, vr, rs[j])
                    ns = jnp.where(m, vs, ss[j])
                    ni = jnp.where(m, vi, is_[j])
                    vr = jnp.where(m, rs[j], vr)
                    vs = jnp.where(m, ss[j], vs)
                    vi = jnp.where(m, is_[j], vi)
                    rs[j], ss[j], is_[j] = nr, ns, ni
            total = ss[0]
            for j in range(1, TOPK):
                total = total + ss[j]
            inv = 1.0 / total
            for j in range(TOPK):
                ow_v[j, pl.ds(t0, LANES)] = ss[j] * inv
                oi_v[j, pl.ds(t0, LANES)] = is_[j]
            return 0

        lax.fori_loop(0, CHUNK // LANES, group_body, 0)
        pltpu.sync_copy(ow_v, ow_hbm.at[:, pl.ds(tok0, CHUNK)])
        pltpu.sync_copy(oi_v, oi_hbm.at[:, pl.ds(tok0, CHUNK)])
        return 0

    lax.fori_loop(0, TPW // CHUNK, chunk_body, 0)


def kernel(x, weight, expert_biases):
    b_col = expert_biases.reshape(N_EXPERTS, 1)
    sc = pl.kernel(
        _sc_topk_kernel,
        mesh=plsc.VectorSubcoreMesh(core_axis_name="c", subcore_axis_name="s"),
        out_type=[
            jax.ShapeDtypeStruct((TOPK, PIPE_TOKENS), jnp.float32),
            jax.ShapeDtypeStruct((TOPK, PIPE_TOKENS), jnp.int32),
        ],
        scratch_types=[
            pltpu.VMEM((N_EXPERTS, CHUNK), jnp.float32),
            pltpu.VMEM((N_EXPERTS, LANES), jnp.float32),
            pltpu.VMEM((TOPK, CHUNK), jnp.float32),
            pltpu.VMEM((TOPK, CHUNK), jnp.int32),
        ],
    )
    routing_t = _tc_routing_t(x, weight, b_col)
    bias_bc = jnp.tile(expert_biases.reshape(N_EXPERTS, 1), (1, LANES))
    ow_t, oi_t = sc(routing_t, bias_bc)
    return ow_t.T, oi_t.T
